# baseline (speedup 1.0000x reference)
import functools

import jax
import jax.numpy as jnp
from jax import lax
from jax.experimental import pallas as pl
from jax.experimental.pallas import tpu as pltpu
from jax.experimental.pallas import tpu_sc as plsc

D = 16
B = 16384


def _make_kernel():
    info = plsc.get_sparse_core_info()
    nw = info.num_cores * info.num_subcores
    bpw = B // nw
    mesh = plsc.VectorSubcoreMesh(core_axis_name="c", subcore_axis_name="s")

    @functools.partial(
        pl.kernel,
        mesh=mesh,
        out_type=jax.ShapeDtypeStruct((B, D), jnp.float32),
        compiler_params=pltpu.CompilerParams(use_tc_tiling_on_sc=False),
        scratch_types=[
            pltpu.VMEM((bpw,), jnp.int32),
            pltpu.VMEM((bpw, D), jnp.float32),
            pltpu.VMEM((bpw, D), jnp.float32),
            pltpu.VMEM((bpw, D), jnp.float32),
            pltpu.SemaphoreType.DMA,
            pltpu.SemaphoreType.DMA,
        ],
    )
    def gm_kernel(k_hbm, eps_hbm, mu_hbm, ls_hbm, out_hbm,
                  idx_v, mu_v, ls_v, eps_v, sem_mu, sem_ls):
        wid = lax.axis_index("s") * info.num_cores + lax.axis_index("c")
        base = wid * bpw
        pltpu.sync_copy(k_hbm.at[pl.ds(base, bpw)], idx_v)
        cp_mu = pltpu.async_copy(mu_hbm.at[idx_v], mu_v, sem_mu)
        cp_ls = pltpu.async_copy(ls_hbm.at[idx_v], ls_v, sem_ls)
        pltpu.sync_copy(eps_hbm.at[pl.ds(base, bpw)], eps_v)
        cp_mu.wait()
        cp_ls.wait()

        def body(i, carry):
            mu_v[i, :] = jnp.exp(ls_v[i, :]) * eps_v[i, :] + mu_v[i, :]
            return carry

        lax.fori_loop(0, bpw, body, 0)
        pltpu.sync_copy(mu_v, out_hbm.at[pl.ds(base, bpw)])

    return gm_kernel


def kernel(k, eps, mu, log_sigma):
    return _make_kernel()(k.astype(jnp.int32), eps, mu, log_sigma)
